# baseline (speedup 1.0000x reference)
import functools

import jax
import jax.numpy as jnp
from jax import lax
from jax.experimental import pallas as pl
from jax.experimental.pallas import tpu as pltpu
from jax.experimental.pallas import tpu_sc as plsc

BATCH = 16384
LABEL_DIM = 32
TEXT_DIM = 768

NC = 2
NS = 16
NW = NC * NS
B_PER_W = BATCH // NW

_MESH = plsc.VectorSubcoreMesh(core_axis_name="c", subcore_axis_name="s")


@functools.partial(
    pl.kernel,
    mesh=_MESH,
    out_type=jax.ShapeDtypeStruct((BATCH, LABEL_DIM), jnp.float32),
    scratch_types=[
        pltpu.VMEM((B_PER_W,), jnp.int32),
        pltpu.VMEM((B_PER_W, LABEL_DIM), jnp.float32),
        pltpu.SemaphoreType.DMA,
    ],
    compiler_params=pltpu.CompilerParams(use_tc_tiling_on_sc=False),
)
def _sc_gather(table_hbm, idx_hbm, out_hbm, idx_v, rows_v, sem):
    wid = lax.axis_index("s") * NC + lax.axis_index("c")
    base = wid * B_PER_W
    pltpu.sync_copy(idx_hbm.at[pl.ds(base, B_PER_W)], idx_v)
    pltpu.async_copy(table_hbm.at[idx_v], rows_v, sem).wait()
    pltpu.sync_copy(rows_v, out_hbm.at[pl.ds(base, B_PER_W)])


def _mm_body(emb_ref, w_ref, b_ref, out_ref):
    out_ref[...] = lax.dot_general(
        emb_ref[...], w_ref[...],
        (((1,), (1,)), ((), ())),
        preferred_element_type=jnp.float32,
    ) + b_ref[...]


BM = 1024


def kernel(labels, label_emb, W, b):
    emb = _sc_gather(label_emb, labels)
    b2d = b.reshape(1, TEXT_DIM)
    out = pl.pallas_call(
        _mm_body,
        grid=(BATCH // BM,),
        in_specs=[
            pl.BlockSpec((BM, LABEL_DIM), lambda i: (i, 0)),
            pl.BlockSpec((TEXT_DIM, LABEL_DIM), lambda i: (0, 0)),
            pl.BlockSpec((1, TEXT_DIM), lambda i: (0, 0)),
        ],
        out_specs=pl.BlockSpec((BM, TEXT_DIM), lambda i: (i, 0)),
        out_shape=jax.ShapeDtypeStruct((BATCH, TEXT_DIM), jnp.float32),
    )(emb, W, b2d)
    return out
